# Initial kernel scaffold; baseline (speedup 1.0000x reference)
import jax
import jax.numpy as jnp
from jax.experimental import pallas as pl


def kernel(x, turn_table, pos_table, civ_table, face_table, card_table, action_table, coin_W, coin_b):
    raise NotImplementedError("write your pallas kernel here")


# speedup vs baseline: 7.6850x; 7.6850x over previous
import functools

import jax
import jax.numpy as jnp
from jax import lax
from jax.experimental import pallas as pl
from jax.experimental.pallas import tpu as pltpu
from jax.experimental.pallas import tpu_sc as plsc

D = 64
L = 16
NC, NS = 2, 16
NW = NC * NS
K = 128

_NT, _NA, _NP, _NV, _NF = 20, 4, 8, 8, 3
_COMBO = _NT * _NA * _NP * _NV * _NF


def _combo_body(turn_ref, action_ref, pos_ref, civ_ref, face_ref, coinb_ref,
                out_ref, *, pos_off):
    def inner_rep(tbl, rep):
        n = tbl.shape[0]
        return jnp.broadcast_to(tbl[:, None, :], (n, rep, D)).reshape(n * rep, D)

    def outer_tile(tbl, times):
        r = tbl.shape[0]
        return jnp.broadcast_to(tbl[None], (times, r, D)).reshape(times * r, D)

    t = inner_rep(turn_ref[:], _NA * _NP * _NV * _NF)
    a = outer_tile(inner_rep(action_ref[:], _NP * _NV * _NF), _NT)
    p = outer_tile(inner_rep(pos_ref[pos_off:pos_off + _NP, :], _NV * _NF),
                   _NT * _NA)
    v = outer_tile(inner_rep(civ_ref[:], _NF), _NT * _NA * _NP)
    f = outer_tile(face_ref[:], _NT * _NA * _NP * _NV)
    out_ref[:] = t + a + p + v + f + coinb_ref[:]


def _build_combo(turn_table, action_table, pos_table, civ_table, face_table,
                 coin_b, pos_off):
    return pl.pallas_call(
        functools.partial(_combo_body, pos_off=pos_off),
        out_shape=jax.ShapeDtypeStruct((_COMBO, D), jnp.float32),
    )(turn_table, action_table, pos_table, civ_table, face_table,
      coin_b.reshape(1, D))


def _sc_body(x_hbm, combo_hbm, card_hbm, w_hbm, out_hbm,
             xbuf, wbuf, cardidx, comboidx, cardbuf, combobuf, outbuf,
             sem1, sem2):
    nt = out_hbm.shape[0] // NW
    nchunks = nt // K
    wid = lax.axis_index("s") * NC + lax.axis_index("c")

    pltpu.sync_copy(w_hbm, wbuf)
    wvec = [[wbuf[pl.ds(c * D + q * L, L)] for q in range(D // L)]
            for c in range(4)]

    iota16 = jnp.arange(L, dtype=jnp.int32)

    def chunk_body(c, carry):
        base = wid * nt + c * K
        pltpu.sync_copy(x_hbm.at[pl.ds(base * 10, K * 10)], xbuf)

        for g in range(K // L):
            off = (g * L + iota16) * 10

            def col(j):
                return plsc.load_gather(xbuf, [off + j]).astype(jnp.int32)

            turn = col(0)
            card = col(1)
            act = col(2)
            pos = col(3)
            civ = col(4)
            face = col(5)
            combo = (((turn * _NA + act) * _NP + pos) * _NV + civ) * _NF + face
            cardidx[pl.ds(g * L, L)] = card
            comboidx[pl.ds(g * L, L)] = combo

        cd = pltpu.async_copy(card_hbm.at[cardidx], cardbuf, sem1)
        td = pltpu.async_copy(combo_hbm.at[comboidx], combobuf, sem2)
        cd.wait()
        td.wait()

        def tok_body(t, carry2):
            coin = [plsc.load_gather(xbuf, [jnp.full((L,), t * 10 + 6 + cc,
                                                     jnp.int32)])
                    for cc in range(4)]
            for q in range(D // L):
                acc = cardbuf[t, pl.ds(q * L, L)] + combobuf[t, pl.ds(q * L, L)]
                for cc in range(4):
                    acc = acc + coin[cc] * wvec[cc][q]
                outbuf[t, pl.ds(q * L, L)] = acc
            return carry2

        lax.fori_loop(0, K, tok_body, 0, unroll=2)
        pltpu.sync_copy(outbuf, out_hbm.at[pl.ds(base, K), :])
        return carry

    lax.fori_loop(0, nchunks, chunk_body, 0)


def _sc_embed(x1d, combo, card_table, coin_w1d, n_tokens):
    mesh = plsc.VectorSubcoreMesh(core_axis_name="c", subcore_axis_name="s")
    return pl.kernel(
        _sc_body,
        out_type=jax.ShapeDtypeStruct((n_tokens, D), jnp.float32),
        mesh=mesh,
        compiler_params=pltpu.CompilerParams(needs_layout_passes=False,
                                             use_tc_tiling_on_sc=False),
        scratch_types=[
            pltpu.VMEM((K * 10,), jnp.float32),
            pltpu.VMEM((4 * D,), jnp.float32),
            pltpu.VMEM((K,), jnp.int32),
            pltpu.VMEM((K,), jnp.int32),
            pltpu.VMEM((K, D), jnp.float32),
            pltpu.VMEM((K, D), jnp.float32),
            pltpu.VMEM((K, D), jnp.float32),
            pltpu.SemaphoreType.DMA,
            pltpu.SemaphoreType.DMA,
        ],
    )(x1d, combo, card_table, coin_w1d)


def kernel(x, turn_table, pos_table, civ_table, face_table, card_table,
           action_table, coin_W, coin_b):
    b, s, feat = x.shape
    assert feat == 10
    n = (s - 6) // 19
    pos_off = {3: 0, 4: 4, 5: 9, 6: 15, 7: 22}[int(n)]
    n_tokens = b * s
    assert n_tokens % (NW * K) == 0

    combo = _build_combo(turn_table, action_table, pos_table, civ_table,
                         face_table, coin_b, pos_off)
    x1d = x.reshape(n_tokens * 10)
    out = _sc_embed(x1d, combo, card_table, coin_W.reshape(4 * D), n_tokens)
    return out.reshape(b, s, D)


# speedup vs baseline: 14.2631x; 1.8560x over previous
import functools

import jax
import jax.numpy as jnp
from jax import lax
from jax.experimental import pallas as pl
from jax.experimental.pallas import tpu as pltpu
from jax.experimental.pallas import tpu_sc as plsc

D = 64
L = 16
NC, NS = 2, 16
NW = NC * NS
K = 128
NG = K // L

_NT, _NA, _NP, _NV, _NF = 20, 4, 8, 8, 3
_COMBO = _NT * _NA * _NP * _NV * _NF


def _combo_body(turn_ref, action_ref, pos_ref, civ_ref, face_ref, coinb_ref,
                out_ref, *, pos_off):
    def inner_rep(tbl, rep):
        n = tbl.shape[0]
        return jnp.broadcast_to(tbl[:, None, :], (n, rep, D)).reshape(n * rep, D)

    def outer_tile(tbl, times):
        r = tbl.shape[0]
        return jnp.broadcast_to(tbl[None], (times, r, D)).reshape(times * r, D)

    t = inner_rep(turn_ref[:], _NA * _NP * _NV * _NF)
    a = outer_tile(inner_rep(action_ref[:], _NP * _NV * _NF), _NT)
    p = outer_tile(inner_rep(pos_ref[pos_off:pos_off + _NP, :], _NV * _NF),
                   _NT * _NA)
    v = outer_tile(inner_rep(civ_ref[:], _NF), _NT * _NA * _NP)
    f = outer_tile(face_ref[:], _NT * _NA * _NP * _NV)
    out_ref[:] = t + a + p + v + f + coinb_ref[:]


def _build_combo(turn_table, action_table, pos_table, civ_table, face_table,
                 coin_b, pos_off):
    return pl.pallas_call(
        functools.partial(_combo_body, pos_off=pos_off),
        out_shape=jax.ShapeDtypeStruct((_COMBO, D), jnp.float32),
    )(turn_table, action_table, pos_table, civ_table, face_table,
      coin_b.reshape(1, D))


def _sc_body(x_hbm, combo_hbm, card_hbm, w_hbm, out_hbm, *s):
    (xb, ci, ti, cb, tb, ob, cn, wbuf) = (
        s[0:2], s[2:4], s[4:6], s[6:8], s[8:10], s[10:12], s[12:14], s[14])
    semx, semc, semt, semo = s[15:17], s[17:19], s[19:21], s[21:23]

    nt = out_hbm.shape[0] // NW
    nchunks = nt // K
    wid = lax.axis_index("s") * NC + lax.axis_index("c")
    tok0 = wid * nt

    pltpu.sync_copy(w_hbm, wbuf)
    wvec = [[wbuf[pl.ds(c * D + q * L, L)] for q in range(D // L)]
            for c in range(4)]

    iota16 = jnp.arange(L, dtype=jnp.int32)

    def bcast_lane(vec, k):
        idx = jnp.full((L, 1), k, jnp.int32)
        dnums = lax.GatherDimensionNumbers(
            offset_dims=(), collapsed_slice_dims=(0,), start_index_map=(0,))
        return lax.gather(vec, idx, dnums, (1,),
                          mode=lax.GatherScatterMode.PROMISE_IN_BOUNDS)

    def fire_x(c, p):
        pltpu.async_copy(x_hbm.at[pl.ds((tok0 + c * K) * 10, K * 10)],
                         xb[p].at[pl.ds(0, K * 10)], semx[p])

    def wait_x(p):
        pltpu.make_async_copy(x_hbm.at[pl.ds(0, K * 10)],
                              xb[p].at[pl.ds(0, K * 10)], semx[p]).wait()

    def extract(p):
        for g in range(NG):
            off = (g * L + iota16) * 10

            def col(j):
                return plsc.load_gather(xb[p], [off + j])

            turn = col(0).astype(jnp.int32)
            card = col(1).astype(jnp.int32)
            act = col(2).astype(jnp.int32)
            pos = col(3).astype(jnp.int32)
            civ = col(4).astype(jnp.int32)
            face = col(5).astype(jnp.int32)
            combo = (((turn * _NA + act) * _NP + pos) * _NV + civ) * _NF + face
            ci[p][pl.ds(g * L, L)] = card
            ti[p][pl.ds(g * L, L)] = combo
            for cc in range(4):
                cn[p][pl.ds(cc * K + g * L, L)] = col(6 + cc)

    def fire_g(p):
        pltpu.async_copy(card_hbm.at[ci[p]], cb[p], semc[p])
        pltpu.async_copy(combo_hbm.at[ti[p]], tb[p], semt[p])

    def wait_g(p):
        pltpu.make_async_copy(card_hbm.at[ci[p]], cb[p], semc[p]).wait()
        pltpu.make_async_copy(combo_hbm.at[ti[p]], tb[p], semt[p]).wait()

    def fire_out(c, p):
        pltpu.async_copy(ob[p], out_hbm.at[pl.ds(tok0 + c * K, K), :], semo[p])

    def wait_out(p):
        pltpu.make_async_copy(ob[p], out_hbm.at[pl.ds(0, K), :],
                              semo[p]).wait()

    def compute(p):
        def grp(g16, carry):
            coinv = [cn[p][pl.ds(cc * K + g16 * L, L)] for cc in range(4)]
            for k in range(L):
                t = g16 * L + k
                acc0 = cb[p][t, pl.ds(0, L)] + tb[p][t, pl.ds(0, L)]
                acc1 = cb[p][t, pl.ds(L, L)] + tb[p][t, pl.ds(L, L)]
                acc2 = cb[p][t, pl.ds(2 * L, L)] + tb[p][t, pl.ds(2 * L, L)]
                acc3 = cb[p][t, pl.ds(3 * L, L)] + tb[p][t, pl.ds(3 * L, L)]
                for cc in range(4):
                    cv = bcast_lane(coinv[cc], k)
                    acc0 = acc0 + cv * wvec[cc][0]
                    acc1 = acc1 + cv * wvec[cc][1]
                    acc2 = acc2 + cv * wvec[cc][2]
                    acc3 = acc3 + cv * wvec[cc][3]
                ob[p][t, pl.ds(0, L)] = acc0
                ob[p][t, pl.ds(L, L)] = acc1
                ob[p][t, pl.ds(2 * L, L)] = acc2
                ob[p][t, pl.ds(3 * L, L)] = acc3
            return carry

        lax.fori_loop(0, NG, grp, 0)

    fire_x(0, 0)
    wait_x(0)
    extract(0)
    fire_g(0)
    fire_x(1, 1)

    def pair(j, carry):
        for sgn in (0, 1):
            c = 2 * j + sgn
            p = sgn
            q = 1 - p

            @pl.when(c + 1 < nchunks)
            def _():
                wait_x(q)
                extract(q)
                fire_g(q)

            @pl.when(c + 2 < nchunks)
            def _():
                fire_x(c + 2, p)

            @pl.when(c < nchunks)
            def _():
                wait_g(p)

            @pl.when(jnp.logical_and(c >= 2, c < nchunks))
            def _():
                wait_out(p)

            @pl.when(c < nchunks)
            def _():
                compute(p)
                fire_out(c, p)

        return carry

    lax.fori_loop(0, (nchunks + 1) // 2, pair, 0)
    wait_out(1)
    wait_out(0)


def _sc_embed(x1d, combo, card_table, coin_w1d, n_tokens):
    mesh = plsc.VectorSubcoreMesh(core_axis_name="c", subcore_axis_name="s")
    dbl = lambda t: [t, t]
    return pl.kernel(
        _sc_body,
        out_type=jax.ShapeDtypeStruct((n_tokens, D), jnp.float32),
        mesh=mesh,
        compiler_params=pltpu.CompilerParams(needs_layout_passes=False,
                                             use_tc_tiling_on_sc=False),
        scratch_types=(
            dbl(pltpu.VMEM((K * 10,), jnp.float32))
            + dbl(pltpu.VMEM((K,), jnp.int32))
            + dbl(pltpu.VMEM((K,), jnp.int32))
            + dbl(pltpu.VMEM((K, D), jnp.float32))
            + dbl(pltpu.VMEM((K, D), jnp.float32))
            + dbl(pltpu.VMEM((K, D), jnp.float32))
            + dbl(pltpu.VMEM((4 * K,), jnp.float32))
            + [pltpu.VMEM((4 * D,), jnp.float32)]
            + [pltpu.SemaphoreType.DMA] * 8
        ),
    )(x1d, combo, card_table, coin_w1d)


def kernel(x, turn_table, pos_table, civ_table, face_table, card_table,
           action_table, coin_W, coin_b):
    b, s, feat = x.shape
    assert feat == 10
    n = (s - 6) // 19
    pos_off = {3: 0, 4: 4, 5: 9, 6: 15, 7: 22}[int(n)]
    n_tokens = b * s
    assert n_tokens % (NW * K) == 0

    combo = _build_combo(turn_table, action_table, pos_table, civ_table,
                         face_table, coin_b, pos_off)
    x1d = x.reshape(n_tokens * 10)
    out = _sc_embed(x1d, combo, card_table, coin_W.reshape(4 * D), n_tokens)
    return out.reshape(b, s, D)
